# baseline (speedup 1.0000x reference)
import functools

import jax
import jax.numpy as jnp
from jax import lax
from jax.experimental import pallas as pl
from jax.experimental.pallas import tpu as pltpu
from jax.experimental.pallas import tpu_sc as plsc

VOCAB = 100000
HID = 768
NSOFT = 20
NC = 2
NS = 16
NW = NC * NS
LANES = 16


def _sc_embed(ids, orig_weight, new_weight, soft_prompt, batch, seq):
    total = batch * seq
    rows_w = total // NW
    ch = 64
    nch = rows_w // ch
    w_per_b = NW // batch
    out_rows = batch * (NSOFT + seq)

    mesh = plsc.VectorSubcoreMesh(core_axis_name="c", subcore_axis_name="s")

    @functools.partial(
        pl.kernel,
        out_type=jax.ShapeDtypeStruct((out_rows + NW, HID), jnp.float32),
        mesh=mesh,
        scratch_types=[
            pltpu.VMEM((rows_w,), jnp.int32),
            pltpu.VMEM((nch, ch), jnp.int32),
            pltpu.VMEM((nch, ch), jnp.int32),
            pltpu.VMEM((nch, ch), jnp.int32),
            pltpu.VMEM((nch, ch), jnp.int32),
            pltpu.VMEM((2, 2 * LANES), jnp.int32),
            pltpu.VMEM((ch, HID), jnp.float32),
            pltpu.VMEM((ch, HID), jnp.float32),
            pltpu.SemaphoreType.DMA,
            pltpu.SemaphoreType.DMA,
        ],
    )
    def body(ids_hbm, orig_hbm, new_hbm, soft_hbm, out_hbm,
             ids_v, idx_a, idx_b, dst_a, dst_b, soft_idx,
             buf_a, buf_b, sem_a, sem_b):
        iota = jnp.arange(LANES, dtype=jnp.int32)
        wid = lax.axis_index("s") * NC + lax.axis_index("c")
        b = wid // w_per_b
        seq_start = (wid % w_per_b) * rows_w
        base = wid * rows_w
        out_base = b * (NSOFT + seq) + NSOFT + seq_start
        trash = out_rows + wid

        @pl.when(wid < batch)
        def _():
            real = iota < (NSOFT - LANES)
            soft_idx[0, pl.ds(0, LANES)] = iota
            soft_idx[0, pl.ds(LANES, LANES)] = jnp.where(real, LANES + iota, 0)
            soft_base = wid * (NSOFT + seq)
            soft_idx[1, pl.ds(0, LANES)] = soft_base + iota
            soft_idx[1, pl.ds(LANES, LANES)] = jnp.where(
                real, soft_base + LANES + iota, trash)
            pltpu.async_copy(soft_hbm.at[soft_idx.at[0]],
                             buf_b.at[pl.ds(0, 2 * LANES)], sem_b).wait()
            pltpu.async_copy(buf_b.at[pl.ds(0, 2 * LANES)],
                             out_hbm.at[soft_idx.at[1]], sem_b).wait()

        pltpu.sync_copy(ids_hbm.at[pl.ds(base, rows_w)], ids_v)

        for p in range(rows_w // LANES):
            v = ids_v[pl.ds(p * LANES, LANES)]
            m = v < VOCAB
            c, q = divmod(p, ch // LANES)
            sl = pl.ds(q * LANES, LANES)
            idx_a[c, sl] = jnp.where(m, v, 0)
            idx_b[c, sl] = jnp.where(m, 0, v - (VOCAB - 1))
            orow = out_base + p * LANES + iota
            dst_a[c, sl] = jnp.where(m, orow, trash)
            dst_b[c, sl] = jnp.where(m, trash, orow)

        for c in range(nch):
            cp_a = pltpu.async_copy(orig_hbm.at[idx_a.at[c]], buf_a, sem_a)
            cp_b = pltpu.async_copy(new_hbm.at[idx_b.at[c]], buf_b, sem_b)
            cp_a.wait()
            scat_a = pltpu.async_copy(buf_a, out_hbm.at[dst_a.at[c]], sem_a)
            cp_b.wait()
            pltpu.async_copy(buf_b, out_hbm.at[dst_b.at[c]], sem_b).wait()
            scat_a.wait()

    out = body(ids, orig_weight, new_weight, soft_prompt)
    return out[:out_rows].reshape(batch, NSOFT + seq, HID)


def kernel(x, orig_weight, new_weight, soft_prompt):
    batch = x.shape[0]
    seq = x.shape[1] - NSOFT
    ids = x[:, NSOFT:].reshape(-1)
    return _sc_embed(ids, orig_weight, new_weight, soft_prompt, batch, seq)


# speedup vs baseline: 1.3534x; 1.3534x over previous
import functools

import jax
import jax.numpy as jnp
from jax import lax
from jax.experimental import pallas as pl
from jax.experimental.pallas import tpu as pltpu
from jax.experimental.pallas import tpu_sc as plsc

VOCAB = 100000
HID = 768
NSOFT = 20
NC = 2
NS = 16
NW = NC * NS
LANES = 16


def _sc_embed(ids, orig_weight, new_weight, soft_prompt, batch, seq):
    total = batch * seq
    rows_w = total // NW
    ch = 32
    nch = rows_w // ch
    w_per_b = NW // batch

    mesh = plsc.VectorSubcoreMesh(core_axis_name="c", subcore_axis_name="s")

    @functools.partial(
        pl.kernel,
        out_type=jax.ShapeDtypeStruct((batch, NSOFT + seq, HID), jnp.float32),
        mesh=mesh,
        scratch_types=[
            pltpu.VMEM((rows_w,), jnp.int32),
            pltpu.VMEM((nch, ch), jnp.int32),
            pltpu.VMEM((nch, ch), jnp.int32),
            pltpu.VMEM((nch, ch), jnp.int32),
            pltpu.VMEM((nch, ch), jnp.int32),
            pltpu.VMEM((2, 2 * LANES), jnp.int32),
            pltpu.VMEM((2, ch, HID), jnp.float32),
            pltpu.VMEM((2, ch, HID), jnp.float32),
            pltpu.SemaphoreType.DMA,
            pltpu.SemaphoreType.DMA,
            pltpu.SemaphoreType.DMA,
            pltpu.SemaphoreType.DMA,
            pltpu.SemaphoreType.DMA,
            pltpu.SemaphoreType.DMA,
            pltpu.SemaphoreType.DMA,
            pltpu.SemaphoreType.DMA,
        ],
    )
    def body(ids_hbm, orig_hbm, new_hbm, soft_hbm, out_hbm,
             ids_v, idx_a, idx_b, dst_a, dst_b, soft_idx, buf_a, buf_b,
             sem_ga0, sem_ga1, sem_sa0, sem_sa1,
             sem_gb0, sem_gb1, sem_sb0, sem_sb1):
        iota = jnp.arange(LANES, dtype=jnp.int32)
        wid = lax.axis_index("c") * NS + lax.axis_index("s")
        b = wid // w_per_b
        w8 = wid % w_per_b
        base = wid * rows_w
        out_b = out_hbm.at[b]
        sem_ga = [sem_ga0, sem_ga1]
        sem_sa = [sem_sa0, sem_sa1]
        sem_gb = [sem_gb0, sem_gb1]
        sem_sb = [sem_sb0, sem_sb1]

        pltpu.sync_copy(ids_hbm.at[pl.ds(base, rows_w)], ids_v)

        for p in range(rows_w // LANES):
            v = ids_v[pl.ds(p * LANES, LANES)]
            m = v < VOCAB
            c, q = divmod(p, ch // LANES)
            sl = pl.ds(q * LANES, LANES)
            idx_a[c, sl] = jnp.where(m, v, 0)
            idx_b[c, sl] = jnp.where(m, 0, v - (VOCAB - 1))
            orow = NSOFT + w8 * rows_w + p * LANES + iota
            dst_a[c, sl] = jnp.where(m, orow, w8)
            dst_b[c, sl] = jnp.where(m, w8, orow)

        def gat(c):
            k = c % 2
            return (
                pltpu.async_copy(orig_hbm.at[idx_a.at[c]],
                                 buf_a.at[k], sem_ga[k]),
                pltpu.async_copy(new_hbm.at[idx_b.at[c]],
                                 buf_b.at[k], sem_gb[k]),
            )

        def scat(c):
            k = c % 2
            return (
                pltpu.async_copy(buf_a.at[k],
                                 out_b.at[dst_a.at[c]], sem_sa[k]),
                pltpu.async_copy(buf_b.at[k],
                                 out_b.at[dst_b.at[c]], sem_sb[k]),
            )

        scats = {}
        gats = {0: gat(0)}
        for c in range(nch):
            if c + 1 < nch:
                if c >= 1:
                    for h in scats.pop(c - 1):
                        h.wait()
                gats[c + 1] = gat(c + 1)
            for h in gats.pop(c):
                h.wait()
            scats[c] = scat(c)

        for c in sorted(scats):
            for h in scats.pop(c):
                h.wait()

        plsc.subcore_barrier()

        @pl.when(w8 == 0)
        def _():
            lo = jnp.minimum(iota, NSOFT - 1)
            hi_half = jnp.minimum(LANES + iota, NSOFT - 1)
            soft_idx[0, pl.ds(0, LANES)] = lo
            soft_idx[0, pl.ds(LANES, LANES)] = hi_half
            soft_idx[1, pl.ds(0, LANES)] = lo
            soft_idx[1, pl.ds(LANES, LANES)] = hi_half
            pltpu.async_copy(soft_hbm.at[soft_idx.at[0]],
                             buf_b.at[0], sem_gb0).wait()
            pltpu.async_copy(buf_b.at[0],
                             out_b.at[soft_idx.at[1]], sem_gb0).wait()

    return body(ids, orig_weight, new_weight, soft_prompt)


def kernel(x, orig_weight, new_weight, soft_prompt):
    batch = x.shape[0]
    seq = x.shape[1] - NSOFT
    ids = x[:, NSOFT:].reshape(-1)
    return _sc_embed(ids, orig_weight, new_weight, soft_prompt, batch, seq)


# speedup vs baseline: 5.3107x; 3.9239x over previous
import functools

import jax
import jax.numpy as jnp
from jax import lax
from jax.experimental import pallas as pl
from jax.experimental.pallas import tpu as pltpu
from jax.experimental.pallas import tpu_sc as plsc

VOCAB = 100000
HID = 768
NSOFT = 20
NC = 2
NS = 16
NW = NC * NS
LANES = 16


def _sc_embed(ids, orig_weight, new_weight, soft_prompt, batch, seq):
    total = batch * seq
    rows_w = total // NW
    ch = 32
    nch = rows_w // ch
    w_per_b = NW // batch

    mesh = plsc.VectorSubcoreMesh(core_axis_name="c", subcore_axis_name="s")

    @functools.partial(
        pl.kernel,
        out_type=jax.ShapeDtypeStruct((batch, NSOFT + seq, HID), jnp.float32),
        mesh=mesh,
        scratch_types=[
            pltpu.VMEM((rows_w,), jnp.int32),
            pltpu.VMEM((nch, ch), jnp.int32),
            pltpu.VMEM((nch, ch), jnp.int32),
            pltpu.VMEM((nch, ch), jnp.int32),
            pltpu.VMEM((nch, ch), jnp.int32),
            pltpu.VMEM((2, 2 * LANES), jnp.int32),
            pltpu.VMEM((2, ch, HID), jnp.float32),
            pltpu.VMEM((2, ch, HID), jnp.float32),
            pltpu.SemaphoreType.DMA,
            pltpu.SemaphoreType.DMA,
            pltpu.SemaphoreType.DMA,
            pltpu.SemaphoreType.DMA,
            pltpu.SemaphoreType.DMA,
            pltpu.SemaphoreType.DMA,
            pltpu.SemaphoreType.DMA,
            pltpu.SemaphoreType.DMA,
        ],
    )
    def body(ids_hbm, orig_hbm, new_hbm, soft_hbm, out_hbm,
             ids_v, idx_a, idx_b, dst_a, dst_b, soft_idx, buf_a, buf_b,
             sem_ga0, sem_ga1, sem_sa0, sem_sa1,
             sem_gb0, sem_gb1, sem_sb0, sem_sb1):
        iota = jnp.arange(LANES, dtype=jnp.int32)
        wid = lax.axis_index("c") * NS + lax.axis_index("s")
        b = wid // w_per_b
        w8 = wid % w_per_b
        base = wid * rows_w
        out_b = out_hbm.at[b]
        sem_ga = [sem_ga0, sem_ga1]
        sem_sa = [sem_sa0, sem_sa1]
        sem_gb = [sem_gb0, sem_gb1]
        sem_sb = [sem_sb0, sem_sb1]

        pltpu.sync_copy(ids_hbm.at[pl.ds(base, rows_w)], ids_v)

        for p in range(rows_w // LANES):
            v = ids_v[pl.ds(p * LANES, LANES)]
            m = v < VOCAB
            c, q = divmod(p, ch // LANES)
            sl = pl.ds(q * LANES, LANES)
            pvec = p * LANES + iota
            spread_b = pvec & 127
            spread_t = pvec & 15
            idx_a[c, sl] = jnp.where(m, v, spread_b)
            idx_b[c, sl] = jnp.where(m, spread_b, v - (VOCAB - 1))
            orow = NSOFT + w8 * rows_w + p * LANES + iota
            dst_a[c, sl] = jnp.where(m, orow, spread_t)
            dst_b[c, sl] = jnp.where(m, spread_t, orow)

        def gat(c):
            k = c % 2
            return (
                pltpu.async_copy(orig_hbm.at[idx_a.at[c]],
                                 buf_a.at[k], sem_ga[k]),
                pltpu.async_copy(new_hbm.at[idx_b.at[c]],
                                 buf_b.at[k], sem_gb[k]),
            )

        def scat(c):
            k = c % 2
            return (
                pltpu.async_copy(buf_a.at[k],
                                 out_b.at[dst_a.at[c]], sem_sa[k]),
                pltpu.async_copy(buf_b.at[k],
                                 out_b.at[dst_b.at[c]], sem_sb[k]),
            )

        scats = {}
        gats = {0: gat(0)}
        for c in range(nch):
            if c + 1 < nch:
                if c >= 1:
                    for h in scats.pop(c - 1):
                        h.wait()
                gats[c + 1] = gat(c + 1)
            for h in gats.pop(c):
                h.wait()
            scats[c] = scat(c)

        for c in sorted(scats):
            for h in scats.pop(c):
                h.wait()

        plsc.subcore_barrier()

        @pl.when(w8 == 0)
        def _():
            lo = jnp.minimum(iota, NSOFT - 1)
            hi_half = jnp.minimum(LANES + iota, NSOFT - 1)
            soft_idx[0, pl.ds(0, LANES)] = lo
            soft_idx[0, pl.ds(LANES, LANES)] = hi_half
            soft_idx[1, pl.ds(0, LANES)] = lo
            soft_idx[1, pl.ds(LANES, LANES)] = hi_half
            pltpu.async_copy(soft_hbm.at[soft_idx.at[0]],
                             buf_b.at[0], sem_gb0).wait()
            pltpu.async_copy(buf_b.at[0],
                             out_b.at[soft_idx.at[1]], sem_gb0).wait()

    return body(ids, orig_weight, new_weight, soft_prompt)


def kernel(x, orig_weight, new_weight, soft_prompt):
    batch = x.shape[0]
    seq = x.shape[1] - NSOFT
    ids = x[:, NSOFT:].reshape(-1)
    return _sc_embed(ids, orig_weight, new_weight, soft_prompt, batch, seq)


# speedup vs baseline: 5.4844x; 1.0327x over previous
import functools

import jax
import jax.numpy as jnp
from jax import lax
from jax.experimental import pallas as pl
from jax.experimental.pallas import tpu as pltpu
from jax.experimental.pallas import tpu_sc as plsc

VOCAB = 100000
HID = 768
NSOFT = 20
NC = 2
NS = 16
NW = NC * NS
LANES = 16


def _sc_embed(ids, orig_weight, new_weight, soft_prompt, batch, seq):
    total = batch * seq
    rows_w = total // NW
    ch = 32
    nch = rows_w // ch
    w_per_b = NW // batch

    mesh = plsc.VectorSubcoreMesh(core_axis_name="c", subcore_axis_name="s")

    @functools.partial(
        pl.kernel,
        out_type=jax.ShapeDtypeStruct((batch, NSOFT + seq, HID), jnp.float32),
        mesh=mesh,
        scratch_types=[
            pltpu.VMEM((rows_w,), jnp.int32),
            pltpu.VMEM((nch, ch), jnp.int32),
            pltpu.VMEM((nch, ch), jnp.int32),
            pltpu.VMEM((nch, ch), jnp.int32),
            pltpu.VMEM((nch, ch), jnp.int32),
            pltpu.VMEM((2, 2 * LANES), jnp.int32),
            pltpu.VMEM((2, ch, HID), jnp.float32),
            pltpu.VMEM((2, ch, HID), jnp.float32),
            pltpu.SemaphoreType.DMA,
            pltpu.SemaphoreType.DMA,
            pltpu.SemaphoreType.DMA,
            pltpu.SemaphoreType.DMA,
            pltpu.SemaphoreType.DMA,
            pltpu.SemaphoreType.DMA,
            pltpu.SemaphoreType.DMA,
            pltpu.SemaphoreType.DMA,
        ],
    )
    def body(ids_hbm, orig_hbm, new_hbm, soft_hbm, out_hbm,
             ids_v, idx_a, idx_b, dst_a, dst_b, soft_idx, buf_a, buf_b,
             sem_ga0, sem_ga1, sem_sa0, sem_sa1,
             sem_gb0, sem_gb1, sem_sb0, sem_sb1):
        iota = jnp.arange(LANES, dtype=jnp.int32)
        wid = lax.axis_index("c") * NS + lax.axis_index("s")
        b = wid // w_per_b
        w8 = wid % w_per_b
        base = wid * rows_w
        out_b = out_hbm.at[b]
        sem_ga = [sem_ga0, sem_ga1]
        sem_sa = [sem_sa0, sem_sa1]
        sem_gb = [sem_gb0, sem_gb1]
        sem_sb = [sem_sb0, sem_sb1]

        pltpu.sync_copy(ids_hbm.at[pl.ds(base, rows_w)], ids_v)

        for p in range(rows_w // LANES):
            v = ids_v[pl.ds(p * LANES, LANES)]
            m = v < VOCAB
            c, q = divmod(p, ch // LANES)
            sl = pl.ds(q * LANES, LANES)
            pvec = p * LANES + iota
            spread_b = pvec & 127
            idx_a[c, sl] = jnp.where(m, v, spread_b)
            idx_b[c, sl] = jnp.where(m, spread_b, v - (VOCAB - 1))
            orow = NSOFT + w8 * rows_w + p * LANES + iota
            trash = (orow + ch) if c < nch - 1 else (pvec & 15)
            dst_a[c, sl] = jnp.where(m, orow, trash)
            dst_b[c, sl] = jnp.where(m, trash, orow)

        def gat(c):
            k = c % 2
            return (
                pltpu.async_copy(orig_hbm.at[idx_a.at[c]],
                                 buf_a.at[k], sem_ga[k]),
                pltpu.async_copy(new_hbm.at[idx_b.at[c]],
                                 buf_b.at[k], sem_gb[k]),
            )

        def scat(c):
            k = c % 2
            return (
                pltpu.async_copy(buf_a.at[k],
                                 out_b.at[dst_a.at[c]], sem_sa[k]),
                pltpu.async_copy(buf_b.at[k],
                                 out_b.at[dst_b.at[c]], sem_sb[k]),
            )

        scats = {}
        gats = {0: gat(0)}
        for c in range(nch):
            if c >= 1:
                for h in scats.pop(c - 1):
                    h.wait()
            if c + 1 < nch:
                gats[c + 1] = gat(c + 1)
            for h in gats.pop(c):
                h.wait()
            scats[c] = scat(c)

        for h in scats.pop(nch - 1):
            h.wait()

        plsc.subcore_barrier()

        @pl.when(w8 == 0)
        def _():
            lo = jnp.minimum(iota, NSOFT - 1)
            hi_half = jnp.minimum(LANES + iota, NSOFT - 1)
            soft_idx[0, pl.ds(0, LANES)] = lo
            soft_idx[0, pl.ds(LANES, LANES)] = hi_half
            soft_idx[1, pl.ds(0, LANES)] = lo
            soft_idx[1, pl.ds(LANES, LANES)] = hi_half
            pltpu.async_copy(soft_hbm.at[soft_idx.at[0]],
                             buf_b.at[0], sem_gb0).wait()
            pltpu.async_copy(buf_b.at[0],
                             out_b.at[soft_idx.at[1]], sem_gb0).wait()

    return body(ids, orig_weight, new_weight, soft_prompt)


def kernel(x, orig_weight, new_weight, soft_prompt):
    batch = x.shape[0]
    seq = x.shape[1] - NSOFT
    ids = x[:, NSOFT:].reshape(-1)
    return _sc_embed(ids, orig_weight, new_weight, soft_prompt, batch, seq)
